# baseline (speedup 1.0000x reference)
import functools

import jax
import jax.numpy as jnp
from jax import lax
from jax.experimental import pallas as pl
from jax.experimental.pallas import tpu as pltpu
from jax.experimental.pallas import tpu_sc as plsc

_NC = 2
_NS = 16
_NW = _NC * _NS

_CHUNK = 32


def _make_gather(V, D, B):
    b_per_w = B // _NW
    nchunk = b_per_w // _CHUNK
    mesh = plsc.VectorSubcoreMesh(core_axis_name="c", subcore_axis_name="s")

    @functools.partial(
        pl.kernel,
        mesh=mesh,
        out_type=jax.ShapeDtypeStruct((B, D), jnp.float32),
        scratch_types=[
            pltpu.VMEM((b_per_w,), jnp.int32),
            pltpu.VMEM((_CHUNK, D), jnp.float32),
            pltpu.VMEM((_CHUNK, D), jnp.float32),
            pltpu.SemaphoreType.DMA,
            pltpu.SemaphoreType.DMA,
            pltpu.SemaphoreType.DMA,
            pltpu.SemaphoreType.DMA,
        ],
    )
    def gather_kernel(table_hbm, idx_hbm, out_hbm, idx_v, buf0, buf1, g0, g1, s0, s1):
        wid = lax.axis_index("s") * _NC + lax.axis_index("c")
        base = wid * b_per_w
        pltpu.sync_copy(idx_hbm.at[pl.ds(base, b_per_w)], idx_v)

        bufs = (buf0, buf1)
        gsem = (g0, g1)
        ssem = (s0, s1)

        def start_gather(c):
            return pltpu.async_copy(
                table_hbm.at[idx_v.at[pl.ds(c * _CHUNK, _CHUNK)]],
                bufs[c % 2],
                gsem[c % 2],
            )

        def start_scatter(c):
            return pltpu.async_copy(
                bufs[c % 2],
                out_hbm.at[pl.ds(base + c * _CHUNK, _CHUNK)],
                ssem[c % 2],
            )

        gd = [None] * nchunk
        sd = [None] * nchunk
        gd[0] = start_gather(0)
        for c in range(nchunk):
            if c + 1 < nchunk:
                if c >= 1:
                    sd[c - 1].wait()
                gd[c + 1] = start_gather(c + 1)
            gd[c].wait()
            sd[c] = start_scatter(c)
        sd[nchunk - 2].wait()
        sd[nchunk - 1].wait()

    return gather_kernel


def kernel(indexs, entity_prompts):
    B = indexs.shape[0]
    V, S, Dm = entity_prompts.shape
    D = S * Dm
    table = entity_prompts.reshape(V, D)
    out = _make_gather(V, D, B)(table, indexs.astype(jnp.int32))
    return out.reshape(B, S, Dm)


# speedup vs baseline: 1.7431x; 1.7431x over previous
import functools

import jax
import jax.numpy as jnp
from jax import lax
from jax.experimental import pallas as pl
from jax.experimental.pallas import tpu as pltpu
from jax.experimental.pallas import tpu_sc as plsc

_NC = 2
_NS = 16
_NW = _NC * _NS

_CHUNK = 16


def _make_gather(V, S, Dm, B):
    b_per_w = B // _NW
    nchunk = b_per_w // _CHUNK
    mesh = plsc.VectorSubcoreMesh(core_axis_name="c", subcore_axis_name="s")

    @functools.partial(
        pl.kernel,
        mesh=mesh,
        out_type=jax.ShapeDtypeStruct((B, S, Dm), jnp.float32),
        scratch_types=[
            pltpu.VMEM((b_per_w,), jnp.int32),
            pltpu.VMEM((_CHUNK, S, Dm), jnp.float32),
            pltpu.VMEM((_CHUNK, S, Dm), jnp.float32),
            pltpu.SemaphoreType.DMA,
            pltpu.SemaphoreType.DMA,
            pltpu.SemaphoreType.DMA,
            pltpu.SemaphoreType.DMA,
        ],
    )
    def gather_kernel(table_hbm, idx_hbm, out_hbm, idx_v, buf0, buf1, g0, g1, s0, s1):
        wid = lax.axis_index("s") * _NC + lax.axis_index("c")
        base = wid * b_per_w
        pltpu.sync_copy(idx_hbm.at[pl.ds(base, b_per_w)], idx_v)

        bufs = (buf0, buf1)
        gsem = (g0, g1)
        ssem = (s0, s1)

        def start_gather(c):
            return pltpu.async_copy(
                table_hbm.at[idx_v.at[pl.ds(c * _CHUNK, _CHUNK)]],
                bufs[c % 2],
                gsem[c % 2],
            )

        def start_scatter(c):
            return pltpu.async_copy(
                bufs[c % 2],
                out_hbm.at[pl.ds(base + c * _CHUNK, _CHUNK)],
                ssem[c % 2],
            )

        gd = [None] * nchunk
        sd = [None] * nchunk
        gd[0] = start_gather(0)
        for c in range(nchunk):
            if c + 1 < nchunk:
                if c >= 1:
                    sd[c - 1].wait()
                gd[c + 1] = start_gather(c + 1)
            gd[c].wait()
            sd[c] = start_scatter(c)
        sd[nchunk - 2].wait()
        sd[nchunk - 1].wait()

    return gather_kernel


def kernel(indexs, entity_prompts):
    B = indexs.shape[0]
    V, S, Dm = entity_prompts.shape
    return _make_gather(V, S, Dm, B)(entity_prompts, indexs.astype(jnp.int32))


# speedup vs baseline: 1.7473x; 1.0024x over previous
import functools

import jax
import jax.numpy as jnp
from jax import lax
from jax.experimental import pallas as pl
from jax.experimental.pallas import tpu as pltpu
from jax.experimental.pallas import tpu_sc as plsc

_NC = 2
_NS = 16
_NW = _NC * _NS

_CHUNK = 16
_PIECES = ((0, 8), (8, 4))


def _make_gather(V, S, Dm, B):
    b_per_w = B // _NW
    nchunk = b_per_w // _CHUNK
    npiece = len(_PIECES)
    mesh = plsc.VectorSubcoreMesh(core_axis_name="c", subcore_axis_name="s")

    scratch = [pltpu.VMEM((b_per_w,), jnp.int32)]
    for _ in range(2):
        for _, sz in _PIECES:
            scratch.append(pltpu.VMEM((_CHUNK, sz, Dm), jnp.float32))
    scratch.extend([pltpu.SemaphoreType.DMA] * (4 * npiece))

    @functools.partial(
        pl.kernel,
        mesh=mesh,
        out_type=jax.ShapeDtypeStruct((B, S, Dm), jnp.float32),
        scratch_types=scratch,
    )
    def gather_kernel(table_hbm, idx_hbm, out_hbm, idx_v, *bufs_and_sems):
        bufs = bufs_and_sems[: 2 * npiece]
        sems = bufs_and_sems[2 * npiece:]
        gsem = sems[: 2 * npiece]
        ssem = sems[2 * npiece:]

        wid = lax.axis_index("s") * _NC + lax.axis_index("c")
        base = wid * b_per_w
        pltpu.sync_copy(idx_hbm.at[pl.ds(base, b_per_w)], idx_v)

        def start_gather(c):
            b = c % 2
            idx = idx_v.at[pl.ds(c * _CHUNK, _CHUNK)]
            return [
                pltpu.async_copy(
                    table_hbm.at[idx, pl.ds(st, sz), :],
                    bufs[b * npiece + p],
                    gsem[b * npiece + p],
                )
                for p, (st, sz) in enumerate(_PIECES)
            ]

        def start_scatter(c):
            b = c % 2
            return [
                pltpu.async_copy(
                    bufs[b * npiece + p],
                    out_hbm.at[pl.ds(base + c * _CHUNK, _CHUNK), pl.ds(st, sz), :],
                    ssem[b * npiece + p],
                )
                for p, (st, sz) in enumerate(_PIECES)
            ]

        gd = [None] * nchunk
        sd = [None] * nchunk
        gd[0] = start_gather(0)
        for c in range(nchunk):
            if c + 1 < nchunk:
                if c >= 1:
                    for d in sd[c - 1]:
                        d.wait()
                gd[c + 1] = start_gather(c + 1)
            for d in gd[c]:
                d.wait()
            sd[c] = start_scatter(c)
        for d in sd[nchunk - 2]:
            d.wait()
        for d in sd[nchunk - 1]:
            d.wait()

    return gather_kernel


def kernel(indexs, entity_prompts):
    B = indexs.shape[0]
    V, S, Dm = entity_prompts.shape
    return _make_gather(V, S, Dm, B)(entity_prompts, indexs.astype(jnp.int32))


# speedup vs baseline: 26.8289x; 15.3544x over previous
import functools

import jax
import jax.numpy as jnp
from jax import lax
from jax.experimental import pallas as pl
from jax.experimental.pallas import tpu as pltpu
from jax.experimental.pallas import tpu_sc as plsc

_NC = 2
_NS = 16
_NW = _NC * _NS


def _make_gather(S, V, Dm, B):
    b_per_w = B // _NW
    mesh = plsc.VectorSubcoreMesh(core_axis_name="c", subcore_axis_name="s")

    @functools.partial(
        pl.kernel,
        mesh=mesh,
        out_type=jax.ShapeDtypeStruct((S, B, Dm), jnp.float32),
        scratch_types=[
            pltpu.VMEM((b_per_w,), jnp.int32),
            pltpu.VMEM((b_per_w, Dm), jnp.float32),
            pltpu.VMEM((b_per_w, Dm), jnp.float32),
            pltpu.SemaphoreType.DMA,
            pltpu.SemaphoreType.DMA,
            pltpu.SemaphoreType.DMA,
            pltpu.SemaphoreType.DMA,
        ],
    )
    def gather_kernel(table_hbm, idx_hbm, out_hbm, idx_v, buf0, buf1, g0, g1, s0, s1):
        wid = lax.axis_index("s") * _NC + lax.axis_index("c")
        base = wid * b_per_w
        pltpu.sync_copy(idx_hbm.at[pl.ds(base, b_per_w)], idx_v)

        bufs = (buf0, buf1)
        gsem = (g0, g1)
        ssem = (s0, s1)

        def start_gather(j):
            return pltpu.async_copy(
                table_hbm.at[j].at[idx_v], bufs[j % 2], gsem[j % 2]
            )

        def start_scatter(j):
            return pltpu.async_copy(
                bufs[j % 2], out_hbm.at[j].at[pl.ds(base, b_per_w)], ssem[j % 2]
            )

        gd = [None] * S
        sd = [None] * S
        gd[0] = start_gather(0)
        for j in range(S):
            if j + 1 < S:
                if j >= 1:
                    sd[j - 1].wait()
                gd[j + 1] = start_gather(j + 1)
            gd[j].wait()
            sd[j] = start_scatter(j)
        sd[S - 2].wait()
        sd[S - 1].wait()

    return gather_kernel


def kernel(indexs, entity_prompts):
    B = indexs.shape[0]
    V, S, Dm = entity_prompts.shape
    table_t = jnp.transpose(entity_prompts, (1, 0, 2))
    out_t = _make_gather(S, V, Dm, B)(table_t, indexs.astype(jnp.int32))
    return jnp.transpose(out_t, (1, 0, 2))


# speedup vs baseline: 27.6564x; 1.0308x over previous
import functools

import jax
import jax.numpy as jnp
from jax import lax
from jax.experimental import pallas as pl
from jax.experimental.pallas import tpu as pltpu
from jax.experimental.pallas import tpu_sc as plsc

_NC = 2
_NS = 16
_NW = _NC * _NS


_NB = 4


def _make_gather(S, V, Dm, B):
    b_per_w = B // _NW
    nb = min(_NB, S)
    mesh = plsc.VectorSubcoreMesh(core_axis_name="c", subcore_axis_name="s")

    scratch = [pltpu.VMEM((b_per_w,), jnp.int32)]
    scratch += [pltpu.VMEM((b_per_w, Dm), jnp.float32) for _ in range(nb)]
    scratch += [pltpu.SemaphoreType.DMA] * (2 * nb)

    @functools.partial(
        pl.kernel,
        mesh=mesh,
        out_type=jax.ShapeDtypeStruct((S, B, Dm), jnp.float32),
        scratch_types=scratch,
    )
    def gather_kernel(table_hbm, idx_hbm, out_hbm, idx_v, *bufs_and_sems):
        bufs = bufs_and_sems[:nb]
        gsem = bufs_and_sems[nb : 2 * nb]
        ssem = bufs_and_sems[2 * nb :]

        wid = lax.axis_index("s") * _NC + lax.axis_index("c")
        base = wid * b_per_w
        pltpu.sync_copy(idx_hbm.at[pl.ds(base, b_per_w)], idx_v)

        def start_gather(j):
            return pltpu.async_copy(
                table_hbm.at[j].at[idx_v], bufs[j % nb], gsem[j % nb]
            )

        def start_scatter(j):
            return pltpu.async_copy(
                bufs[j % nb], out_hbm.at[j].at[pl.ds(base, b_per_w)], ssem[j % nb]
            )

        gd = [None] * S
        sd = [None] * S
        for j in range(nb):
            gd[j] = start_gather(j)
        for j in range(S):
            if j >= 1 and j - 1 + nb < S:
                sd[j - 1].wait()
                gd[j - 1 + nb] = start_gather(j - 1 + nb)
            gd[j].wait()
            sd[j] = start_scatter(j)
        for j in range(max(0, S - nb), S):
            if sd[j] is not None:
                sd[j].wait()

    return gather_kernel


def kernel(indexs, entity_prompts):
    B = indexs.shape[0]
    V, S, Dm = entity_prompts.shape
    table_t = jnp.transpose(entity_prompts, (1, 0, 2))
    out_t = _make_gather(S, V, Dm, B)(table_t, indexs.astype(jnp.int32))
    return jnp.transpose(out_t, (1, 0, 2))


# speedup vs baseline: 28.0272x; 1.0134x over previous
import functools

import jax
import jax.numpy as jnp
from jax import lax
from jax.experimental import pallas as pl
from jax.experimental.pallas import tpu as pltpu
from jax.experimental.pallas import tpu_sc as plsc

_NC = 2
_NS = 16
_NW = _NC * _NS


_NB = 6


def _make_gather(S, V, Dm, B):
    b_per_w = B // _NW
    nb = min(_NB, S)
    mesh = plsc.VectorSubcoreMesh(core_axis_name="c", subcore_axis_name="s")

    scratch = [pltpu.VMEM((b_per_w,), jnp.int32)]
    scratch += [pltpu.VMEM((b_per_w, Dm), jnp.float32) for _ in range(nb)]
    scratch += [pltpu.SemaphoreType.DMA] * (2 * nb)

    @functools.partial(
        pl.kernel,
        mesh=mesh,
        out_type=jax.ShapeDtypeStruct((S, B, Dm), jnp.float32),
        scratch_types=scratch,
    )
    def gather_kernel(table_hbm, idx_hbm, out_hbm, idx_v, *bufs_and_sems):
        bufs = bufs_and_sems[:nb]
        gsem = bufs_and_sems[nb : 2 * nb]
        ssem = bufs_and_sems[2 * nb :]

        wid = lax.axis_index("s") * _NC + lax.axis_index("c")
        base = wid * b_per_w
        pltpu.sync_copy(idx_hbm.at[pl.ds(base, b_per_w)], idx_v)

        def start_gather(j):
            return pltpu.async_copy(
                table_hbm.at[j].at[idx_v], bufs[j % nb], gsem[j % nb]
            )

        def start_scatter(j):
            return pltpu.async_copy(
                bufs[j % nb], out_hbm.at[j].at[pl.ds(base, b_per_w)], ssem[j % nb]
            )

        gd = [None] * S
        sd = [None] * S
        for j in range(nb):
            gd[j] = start_gather(j)
        for j in range(S):
            if j >= 1 and j - 1 + nb < S:
                sd[j - 1].wait()
                gd[j - 1 + nb] = start_gather(j - 1 + nb)
            gd[j].wait()
            sd[j] = start_scatter(j)
        for j in range(max(0, S - nb), S):
            if sd[j] is not None:
                sd[j].wait()

    return gather_kernel


def kernel(indexs, entity_prompts):
    B = indexs.shape[0]
    V, S, Dm = entity_prompts.shape
    table_t = jnp.transpose(entity_prompts, (1, 0, 2))
    out_t = _make_gather(S, V, Dm, B)(table_t, indexs.astype(jnp.int32))
    return jnp.transpose(out_t, (1, 0, 2))
